# baseline (speedup 1.0000x reference)
import functools

import jax
import jax.numpy as jnp
from jax import lax
from jax.experimental import pallas as pl
from jax.experimental.pallas import tpu as pltpu
from jax.experimental.pallas import tpu_sc as plsc

_NW = 32
_LANES = 16


def _lpe_body(L, B, nblk_total, uv_hbm, mu_hbm, mv_hbm, out_hbm,
              uv_v, out_v, mu_v, mv_v):
    cid = lax.axis_index("c")
    sid = lax.axis_index("s")
    wid = sid * 2 + cid
    vpb = B // _LANES
    kmax = (nblk_total + _NW - 1) // _NW

    pltpu.sync_copy(mu_hbm, mu_v)
    pltpu.sync_copy(mv_hbm, mv_v)

    iota = lax.iota(jnp.int32, _LANES)
    z16 = jnp.zeros((_LANES,), jnp.int32)
    o16 = jnp.ones((_LANES,), jnp.int32)
    c2 = jnp.full((_LANES,), 2, jnp.int32)
    c3 = jnp.full((_LANES,), 3, jnp.int32)
    maxi = jnp.full((_LANES,), L - 1, jnp.int32)
    fscale = jnp.float32(L - 1)

    def lerp2(tab_v, i0, i1, w):
        a0 = plsc.load_gather(tab_v, [z16, i0])
        a1 = plsc.load_gather(tab_v, [z16, i1])
        b0 = plsc.load_gather(tab_v, [o16, i0])
        b1 = plsc.load_gather(tab_v, [o16, i1])
        return a0 + w * (a1 - a0), b0 + w * (b1 - b0)

    def vec_body(jj):
        pt = jj * _LANES + iota
        u = plsc.load_gather(uv_v, [pt, z16])
        v = plsc.load_gather(uv_v, [pt, o16])

        tu = (u + 1.0) * 0.5 * fscale
        tv = (v + 1.0) * 0.5 * fscale
        iu0 = tu.astype(jnp.int32)
        iv0 = tv.astype(jnp.int32)
        wu = tu - iu0.astype(jnp.float32)
        wv = tv - iv0.astype(jnp.float32)
        iu0 = jnp.minimum(jnp.maximum(iu0, z16), maxi)
        iv0 = jnp.minimum(jnp.maximum(iv0, z16), maxi)
        iu1 = jnp.minimum(iu0 + 1, maxi)
        iv1 = jnp.minimum(iv0 + 1, maxi)

        mu0, mu1 = lerp2(mu_v, iu0, iu1, wu)
        mv0, mv1 = lerp2(mv_v, iv0, iv1, wv)

        plsc.store_scatter(out_v, [pt, z16], mu0)
        plsc.store_scatter(out_v, [pt, o16], mu1)
        plsc.store_scatter(out_v, [pt, c2], mv0)
        plsc.store_scatter(out_v, [pt, c3], mv1)

    def blk_body(k, _):
        b = k * _NW + wid

        @pl.when(b < nblk_total)
        def _():
            base = b * B
            pltpu.sync_copy(uv_hbm.at[pl.ds(base, B)], uv_v)
            plsc.parallel_loop(0, vpb, unroll=8)(vec_body)
            pltpu.sync_copy(out_v, out_hbm.at[pl.ds(base, B)])

        return 0

    lax.fori_loop(0, kmax, blk_body, 0)


def kernel(uv, m_u, m_v):
    N = uv.shape[0]
    L = m_u.shape[1]
    B = 2000
    nblk_total = N // B

    mesh = plsc.VectorSubcoreMesh(core_axis_name="c", subcore_axis_name="s")
    f = pl.kernel(
        functools.partial(_lpe_body, L, B, nblk_total),
        out_type=jax.ShapeDtypeStruct((N, 4), jnp.float32),
        mesh=mesh,
        compiler_params=pltpu.CompilerParams(
            needs_layout_passes=False, use_tc_tiling_on_sc=False),
        scratch_types=[
            pltpu.VMEM((B, 2), jnp.float32),
            pltpu.VMEM((B, 4), jnp.float32),
            pltpu.VMEM(m_u.shape, jnp.float32),
            pltpu.VMEM(m_v.shape, jnp.float32),
        ],
    )
    return f(uv, m_u, m_v)


# speedup vs baseline: 9.9665x; 9.9665x over previous
import functools

import jax
import jax.numpy as jnp
from jax import lax
from jax.experimental import pallas as pl
from jax.experimental.pallas import tpu as pltpu
from jax.experimental.pallas import tpu_sc as plsc

_NW = 32
_LANES = 16


def _lpe_body(L, B, nblk_total, u_hbm, v_hbm, mu_hbm, mv_hbm,
              o0_hbm, o1_hbm, o2_hbm, o3_hbm,
              u_v, v_v, o0_v, o1_v, o2_v, o3_v, mu_v, mv_v):
    cid = lax.axis_index("c")
    sid = lax.axis_index("s")
    wid = sid * 2 + cid
    vpb = B // _LANES
    kmax = (nblk_total + _NW - 1) // _NW

    pltpu.sync_copy(mu_hbm, mu_v)
    pltpu.sync_copy(mv_hbm, mv_v)

    z16 = jnp.zeros((_LANES,), jnp.int32)
    maxi = jnp.full((_LANES,), L - 1, jnp.int32)
    cL = jnp.full((_LANES,), L, jnp.int32)
    fscale = jnp.float32(L - 1)

    def lerp2(tab_v, i0, i1, w):
        a0 = plsc.load_gather(tab_v, [i0])
        a1 = plsc.load_gather(tab_v, [i1])
        b0 = plsc.load_gather(tab_v, [i0 + cL])
        b1 = plsc.load_gather(tab_v, [i1 + cL])
        return a0 + w * (a1 - a0), b0 + w * (b1 - b0)

    def vec_body(jj):
        sl = pl.ds(jj * _LANES, _LANES)
        u = u_v[sl]
        v = v_v[sl]

        tu = (u + 1.0) * 0.5 * fscale
        tv = (v + 1.0) * 0.5 * fscale
        iu0 = tu.astype(jnp.int32)
        iv0 = tv.astype(jnp.int32)
        wu = tu - iu0.astype(jnp.float32)
        wv = tv - iv0.astype(jnp.float32)
        iu0 = jnp.minimum(jnp.maximum(iu0, z16), maxi)
        iv0 = jnp.minimum(jnp.maximum(iv0, z16), maxi)
        iu1 = jnp.minimum(iu0 + 1, maxi)
        iv1 = jnp.minimum(iv0 + 1, maxi)

        mu0, mu1 = lerp2(mu_v, iu0, iu1, wu)
        mv0, mv1 = lerp2(mv_v, iv0, iv1, wv)

        o0_v[sl] = mu0
        o1_v[sl] = mu1
        o2_v[sl] = mv0
        o3_v[sl] = mv1

    def blk_body(k, _):
        b = k * _NW + wid

        @pl.when(b < nblk_total)
        def _():
            base = b * B
            sl = pl.ds(base, B)
            pltpu.sync_copy(u_hbm.at[sl], u_v)
            pltpu.sync_copy(v_hbm.at[sl], v_v)
            plsc.parallel_loop(0, vpb, unroll=8)(vec_body)
            pltpu.sync_copy(o0_v, o0_hbm.at[sl])
            pltpu.sync_copy(o1_v, o1_hbm.at[sl])
            pltpu.sync_copy(o2_v, o2_hbm.at[sl])
            pltpu.sync_copy(o3_v, o3_hbm.at[sl])

        return 0

    lax.fori_loop(0, kmax, blk_body, 0)


def kernel(uv, m_u, m_v):
    N = uv.shape[0]
    L = m_u.shape[1]
    B = 2000
    nblk_total = N // B

    mesh = plsc.VectorSubcoreMesh(core_axis_name="c", subcore_axis_name="s")
    plane = jax.ShapeDtypeStruct((N,), jnp.float32)
    f = pl.kernel(
        functools.partial(_lpe_body, L, B, nblk_total),
        out_type=(plane, plane, plane, plane),
        mesh=mesh,
        compiler_params=pltpu.CompilerParams(
            needs_layout_passes=False, use_tc_tiling_on_sc=False),
        scratch_types=[
            pltpu.VMEM((B,), jnp.float32),
            pltpu.VMEM((B,), jnp.float32),
            pltpu.VMEM((B,), jnp.float32),
            pltpu.VMEM((B,), jnp.float32),
            pltpu.VMEM((B,), jnp.float32),
            pltpu.VMEM((B,), jnp.float32),
            pltpu.VMEM((2 * L,), jnp.float32),
            pltpu.VMEM((2 * L,), jnp.float32),
        ],
    )
    o0, o1, o2, o3 = f(uv[:, 0], uv[:, 1],
                       m_u.reshape(2 * L), m_v.reshape(2 * L))
    return jnp.stack([o0, o1, o2, o3], axis=1)


# speedup vs baseline: 10.6085x; 1.0644x over previous
import functools

import jax
import jax.numpy as jnp
from jax import lax
from jax.experimental import pallas as pl
from jax.experimental.pallas import tpu as pltpu
from jax.experimental.pallas import tpu_sc as plsc

_NW = 32
_LANES = 16


def _lpe_body(L, B, nblk_total, u_hbm, v_hbm, mu_hbm, mv_hbm,
              o0_hbm, o1_hbm, o2_hbm, o3_hbm,
              u_v, v_v, o_v, mu_v, mv_v, sem_in, sem_out, sem_tab):
    cid = lax.axis_index("c")
    sid = lax.axis_index("s")
    wid = sid * 2 + cid
    vpb = B // _LANES
    kmax = (nblk_total + _NW - 1) // _NW

    tab_cp = (pltpu.async_copy(mu_hbm, mu_v, sem_tab),
              pltpu.async_copy(mv_hbm, mv_v, sem_tab))

    z16 = jnp.zeros((_LANES,), jnp.int32)
    maxi = jnp.full((_LANES,), L - 1, jnp.int32)
    cL = jnp.full((_LANES,), L, jnp.int32)
    fscale = jnp.float32(L - 1)

    def in_copies(b, buf):
        sl = pl.ds(b * B, B)
        return (pltpu.async_copy(u_hbm.at[sl], u_v.at[buf], sem_in[buf]),
                pltpu.async_copy(v_hbm.at[sl], v_v.at[buf], sem_in[buf]))

    def out_copies(b, buf):
        sl = pl.ds(b * B, B)
        return (pltpu.async_copy(o_v.at[buf, 0], o0_hbm.at[sl], sem_out[buf]),
                pltpu.async_copy(o_v.at[buf, 1], o1_hbm.at[sl], sem_out[buf]),
                pltpu.async_copy(o_v.at[buf, 2], o2_hbm.at[sl], sem_out[buf]),
                pltpu.async_copy(o_v.at[buf, 3], o3_hbm.at[sl], sem_out[buf]))

    def lerp2(tab_v, i0, i1, w):
        a0 = plsc.load_gather(tab_v, [i0])
        a1 = plsc.load_gather(tab_v, [i1])
        b0 = plsc.load_gather(tab_v, [i0 + cL])
        b1 = plsc.load_gather(tab_v, [i1 + cL])
        return a0 + w * (a1 - a0), b0 + w * (b1 - b0)

    def make_vec_body(buf):
        def vec_body(jj):
            sl = pl.ds(jj * _LANES, _LANES)
            u = u_v[buf, sl]
            v = v_v[buf, sl]

            tu = (u + 1.0) * 0.5 * fscale
            tv = (v + 1.0) * 0.5 * fscale
            iu0 = tu.astype(jnp.int32)
            iv0 = tv.astype(jnp.int32)
            wu = tu - iu0.astype(jnp.float32)
            wv = tv - iv0.astype(jnp.float32)
            iu0 = jnp.minimum(jnp.maximum(iu0, z16), maxi)
            iv0 = jnp.minimum(jnp.maximum(iv0, z16), maxi)
            iu1 = jnp.minimum(iu0 + 1, maxi)
            iv1 = jnp.minimum(iv0 + 1, maxi)

            mu0, mu1 = lerp2(mu_v, iu0, iu1, wu)
            mv0, mv1 = lerp2(mv_v, iv0, iv1, wv)

            o_v[buf, 0, sl] = mu0
            o_v[buf, 1, sl] = mu1
            o_v[buf, 2, sl] = mv0
            o_v[buf, 3, sl] = mv1
        return vec_body

    @pl.when(wid < nblk_total)
    def _():
        in_copies(wid, 0)

    tab_cp[0].wait()
    tab_cp[1].wait()

    for k in range(kmax):
        cur = k % 2
        b = k * _NW + wid

        if k + 1 < kmax:
            bn = (k + 1) * _NW + wid

            @pl.when(bn < nblk_total)
            def _(bn=bn, nxt=1 - cur):
                in_copies(bn, nxt)

        @pl.when(b < nblk_total)
        def _(k=k, b=b, cur=cur):
            sl = pl.ds(b * B, B)
            pltpu.make_async_copy(u_hbm.at[sl], u_v.at[cur], sem_in[cur]).wait()
            pltpu.make_async_copy(v_hbm.at[sl], v_v.at[cur], sem_in[cur]).wait()
            if k >= 2:
                bp = (k - 2) * _NW + wid
                slp = pl.ds(bp * B, B)
                for i, oh in enumerate((o0_hbm, o1_hbm, o2_hbm, o3_hbm)):
                    pltpu.make_async_copy(
                        o_v.at[cur, i], oh.at[slp], sem_out[cur]).wait()
            plsc.parallel_loop(0, vpb, unroll=8)(make_vec_body(cur))
            out_copies(b, cur)

    for k in (kmax - 2, kmax - 1):
        if k >= 0:
            b = k * _NW + wid

            @pl.when(b < nblk_total)
            def _(k=k, b=b):
                sl = pl.ds(b * B, B)
                for i, oh in enumerate((o0_hbm, o1_hbm, o2_hbm, o3_hbm)):
                    pltpu.make_async_copy(
                        o_v.at[k % 2, i], oh.at[sl], sem_out[k % 2]).wait()


def kernel(uv, m_u, m_v):
    N = uv.shape[0]
    L = m_u.shape[1]
    B = 4000
    nblk_total = N // B

    mesh = plsc.VectorSubcoreMesh(core_axis_name="c", subcore_axis_name="s")
    plane = jax.ShapeDtypeStruct((N,), jnp.float32)
    f = pl.kernel(
        functools.partial(_lpe_body, L, B, nblk_total),
        out_type=(plane, plane, plane, plane),
        mesh=mesh,
        compiler_params=pltpu.CompilerParams(
            needs_layout_passes=False, use_tc_tiling_on_sc=False),
        scratch_types=[
            pltpu.VMEM((2, B), jnp.float32),
            pltpu.VMEM((2, B), jnp.float32),
            pltpu.VMEM((2, 4, B), jnp.float32),
            pltpu.VMEM((2 * L,), jnp.float32),
            pltpu.VMEM((2 * L,), jnp.float32),
            (pltpu.SemaphoreType.DMA, pltpu.SemaphoreType.DMA),
            (pltpu.SemaphoreType.DMA, pltpu.SemaphoreType.DMA),
            pltpu.SemaphoreType.DMA,
        ],
    )
    o0, o1, o2, o3 = f(uv[:, 0], uv[:, 1],
                       m_u.reshape(2 * L), m_v.reshape(2 * L))
    return jnp.stack([o0, o1, o2, o3], axis=1)
